# baseline (speedup 1.0000x reference)
import jax
import jax.numpy as jnp
from jax import lax
from jax.experimental import pallas as pl
from jax.experimental.pallas import tpu as pltpu
from jax.experimental.pallas import tpu_sc as plsc

B = 4
S = 4096
D = 1024
NC = 2
NS = 16
LN = 16
TPB = NS // (B // NC)
ROWS_PER_TILE = (B * S) // (NC * NS)
FB = 16
N_FILL_CHUNKS = ROWS_PER_TILE // FB
SB = 512
LW = 128


def _count_body(x_ref, lens_ref):
    j = pl.program_id(0)
    sb = pl.program_id(1)

    @pl.when((j == 0) & (sb == 0))
    def _():
        lens_ref[...] = jnp.zeros((B, LW), jnp.int32)

    xb = x_ref[0]
    rs = jnp.sum(xb, axis=1)
    cnt = jnp.sum((rs != 0.0).astype(jnp.int32))
    row_iota = lax.broadcasted_iota(jnp.int32, (B, LW), 0)
    lens_ref[...] += jnp.where(row_iota == j, cnt, 0)


def _fill_body(table_hbm, ivec_hbm, zeros_hbm, lens_hbm, out_hbm,
               icode_buf, zero_buf, ivec_v, lens_v, sem_a, sem_c):
    c = lax.axis_index("c")
    s = lax.axis_index("s")
    batch = c * (B // NC) + s // TPB
    k = s % TPB
    seq_base = k * ROWS_PER_TILE

    pltpu.sync_copy(ivec_hbm, ivec_v)
    i_val = ivec_v[...][0]
    prep_cps = [
        pltpu.async_copy(table_hbm.at[pl.ds(i_val, 1), :],
                         icode_buf.at[pl.ds(f, 1), :], sem_c)
        for f in range(FB)
    ]
    prep_cps.append(pltpu.async_copy(zeros_hbm, zero_buf, sem_c))

    pltpu.sync_copy(lens_hbm.at[batch], lens_v)
    lens = lens_v[pl.ds(0, LN)][0]
    nf = jnp.clip(lens - seq_base, 0, ROWS_PER_TILE)
    fc = nf // FB
    cc_ = (nf + FB - 1) // FB

    for cp in prep_cps:
        cp.wait()

    fired = []
    for t in range(N_FILL_CHUNKS):
        dst = batch * S + seq_base + t * FB
        icp = pltpu.make_async_copy(
            icode_buf, out_hbm.at[pl.ds(dst, FB), :], sem_a)
        zcp = pltpu.make_async_copy(
            zero_buf, out_hbm.at[pl.ds(dst, FB), :], sem_a)
        tt = jnp.int32(t)

        @pl.when(tt < fc)
        def _():
            icp.start()

        @pl.when(tt >= cc_)
        def _():
            zcp.start()

        fired.append((tt, icp, zcp))

    def fill_row(rr, _):
        pltpu.sync_copy(icode_buf.at[pl.ds(0, 1), :],
                        out_hbm.at[pl.ds(batch * S + rr, 1), :])
        return 0

    def zero_row(rr, _):
        pltpu.sync_copy(zero_buf.at[pl.ds(0, 1), :],
                        out_hbm.at[pl.ds(batch * S + rr, 1), :])
        return 0

    lax.fori_loop(fc * FB, nf, fill_row, 0)
    lax.fori_loop(nf, cc_ * FB, zero_row, 0)

    for tt, icp, zcp in fired:
        @pl.when(tt < fc)
        def _():
            icp.wait()

        @pl.when(tt >= cc_)
        def _():
            zcp.wait()


@jax.jit
def _run(x, table, ivec, zeros_src):
    lens = pl.pallas_call(
        _count_body,
        grid=(B, S // SB),
        in_specs=[pl.BlockSpec((1, SB, D), lambda j, sb: (j, sb, 0))],
        out_specs=pl.BlockSpec((B, LW), lambda j, sb: (0, 0)),
        out_shape=jax.ShapeDtypeStruct((B, LW), jnp.int32),
    )(x)

    mesh = plsc.VectorSubcoreMesh(core_axis_name="c", subcore_axis_name="s")
    out = pl.kernel(
        _fill_body,
        out_type=jax.ShapeDtypeStruct((B * S, D), jnp.float32),
        mesh=mesh,
        scratch_types=[
            pltpu.VMEM((FB, D), jnp.float32),
            pltpu.VMEM((FB, D), jnp.float32),
            pltpu.VMEM((LN,), jnp.int32),
            pltpu.VMEM((LW,), jnp.int32),
            pltpu.SemaphoreType.DMA,
            pltpu.SemaphoreType.DMA,
        ],
    )(table, ivec, zeros_src, lens)
    return out.reshape(B, S, D)


def kernel(x, table, i):
    ivec = jnp.full((LN,), i, jnp.int32)
    zeros_src = jnp.zeros((FB, D), jnp.float32)
    return _run(x, table, ivec, zeros_src)
